# baseline (speedup 1.0000x reference)
import dataclasses
import functools

import jax
import jax.numpy as jnp
from jax import lax
from jax.experimental import pallas as pl
from jax.experimental.pallas import tpu as pltpu
from jax.experimental.pallas import tpu_sc as plsc

N = 10000
D = 256
E = 160000
NTILES = 32
RPT = 320
NPAD = NTILES * RPT
CHUNK = 1600
NCHUNKS = E // CHUNK
G = 64
ESL = E // NTILES
ESLPAD = 5008
NROW_LAST = N - (NTILES - 1) * RPT
SL = NPAD // 16

_mesh = plsc.VectorSubcoreMesh(core_axis_name="c", subcore_axis_name="s")

_cp = pltpu.CompilerParams()
if "needs_layout_passes" in pltpu.CompilerParams.__dataclass_fields__:
    _cp = dataclasses.replace(_cp, needs_layout_passes=False)


@functools.partial(
    pl.kernel,
    out_type=(
        jax.ShapeDtypeStruct((2, NPAD), jnp.float32),
        jax.ShapeDtypeStruct((NPAD, D // 2), jnp.int32),
    ),
    mesh=_mesh,
    scratch_types=[
        pltpu.VMEM((ESLPAD,), jnp.int32),
        pltpu.VMEM((NPAD,), jnp.float32),
        pltpu.VMEM_SHARED((16, NPAD), jnp.float32),
        pltpu.VMEM((SL,), jnp.float32),
        pltpu.VMEM((G, D), jnp.float32),
        pltpu.VMEM((G, D // 2), jnp.int32),
    ],
    compiler_params=_cp,
)
def _deg_kernel(row_hbm, x_hbm, degp_hbm, xpk_hbm, rows_v, hist_v, shared,
                red_v, stage, stage_i):
    cid = lax.axis_index("c")
    sid = lax.axis_index("s")
    wid = sid * 2 + cid

    def _pack_rows(nrows):
        def body(r, carry):
            for j in range(0, D // 2, 16):
                a = plsc.bitcast(stage[r, pl.ds(j, 16)], jnp.int32)
                b = plsc.bitcast(stage[r, pl.ds(j + D // 2, 16)],
                                 jnp.int32)
                a = (a + 0x8000) >> 16
                b = (b + 0x8000) & jnp.int32(-65536)
                stage_i[r, pl.ds(j, 16)] = (a & 0xFFFF) | b
            return carry

        lax.fori_loop(0, nrows, body, 0)

    @pl.loop(0, RPT, step=G)
    def _(gi):
        s0 = wid * RPT + gi

        @pl.when(s0 + G <= N)
        def _():
            pltpu.sync_copy(x_hbm.at[pl.ds(s0, G)], stage)
            _pack_rows(G)
            pltpu.sync_copy(stage_i, xpk_hbm.at[pl.ds(s0, G)])

        @pl.when((s0 < N) & (s0 + G > N))
        def _():
            nr = N % G
            pltpu.sync_copy(x_hbm.at[pl.ds(s0, nr)],
                            stage.at[pl.ds(0, nr)])
            _pack_rows(nr)
            pltpu.sync_copy(stage_i.at[pl.ds(0, nr)],
                            xpk_hbm.at[pl.ds(s0, nr)])

    zf = jnp.zeros((16,), jnp.float32)

    @pl.loop(0, NPAD, step=16)
    def _(i):
        hist_v[pl.ds(i, 16)] = zf

    pltpu.sync_copy(row_hbm.at[pl.ds(wid * ESL, ESL)],
                    rows_v.at[pl.ds(0, ESL)])

    ones = jnp.ones((16,), jnp.float32)

    ESL_FULL = (ESL // 16) * 16

    @pl.loop(0, ESL_FULL, step=16)
    def _(i):
        idx = rows_v[pl.ds(i, 16)]
        plsc.addupdate_scatter(hist_v, [idx], ones)

    lane = lax.broadcasted_iota(jnp.int32, (16,), 0)
    tidx = rows_v[pl.ds(ESL_FULL, 16)]
    plsc.addupdate_scatter(hist_v, [tidx], ones, mask=lane < (ESL - ESL_FULL))

    pltpu.sync_copy(hist_v, shared.at[sid])
    plsc.subcore_barrier()

    @pl.loop(0, SL, step=16)
    def _(i):
        red_v[pl.ds(i, 16)] = zf

    @pl.loop(0, 16)
    def _(t):
        pltpu.sync_copy(shared.at[t, pl.ds(sid * SL, SL)],
                        hist_v.at[pl.ds(0, SL)])

        @pl.loop(0, SL, step=16)
        def _(i):
            red_v[pl.ds(i, 16)] = red_v[pl.ds(i, 16)] + hist_v[pl.ds(i, 16)]

    pltpu.sync_copy(red_v, degp_hbm.at[cid, pl.ds(sid * SL, SL)])


@functools.partial(
    pl.kernel,
    out_type=jax.ShapeDtypeStruct((N, D), jnp.float32),
    mesh=_mesh,
    scratch_types=[
        pltpu.VMEM((RPT, D), jnp.float32),
        pltpu.VMEM((NPAD,), jnp.float32),
        pltpu.VMEM((CHUNK,), jnp.int32),
        pltpu.VMEM((CHUNK,), jnp.int32),
        pltpu.VMEM((CHUNK + G,), jnp.int32),
        pltpu.VMEM((CHUNK + G,), jnp.int32),
        pltpu.VMEM((G,), jnp.int32),
        pltpu.VMEM((G, D // 2), jnp.int32),
        pltpu.VMEM((1024,), jnp.float32),
        pltpu.VMEM((1024,), jnp.float32),
        pltpu.SemaphoreType.DMA,
    ],
    compiler_params=_cp,
)
def _prop_kernel(x_hbm, row_hbm, col_hbm, degp_hbm, xpk_hbm, out_hbm,
                 acc, dis, rowb, colb, frloc, fcol, idxb,
                 stage_i, tmp0, tmp1, gsem):
    cid = lax.axis_index("c")
    sid = lax.axis_index("s")
    wid = sid * 2 + cid
    base = wid * RPT

    @pl.loop(0, NPAD, step=1024)
    def _(off):
        pltpu.sync_copy(degp_hbm.at[0, pl.ds(off, 1024)], tmp0)
        pltpu.sync_copy(degp_hbm.at[1, pl.ds(off, 1024)], tmp1)

        @pl.loop(0, 1024, step=16)
        def _(i):
            d = tmp0[pl.ds(i, 16)] + tmp1[pl.ds(i, 16)]
            bits = plsc.bitcast(d, jnp.int32)
            bits = 0x5F3759DF - (bits >> 1)
            y = plsc.bitcast(bits, jnp.float32)
            hd = -0.5 * d
            y = y * (1.5 + hd * y * y)
            y = y * (1.5 + hd * y * y)
            y = y * (1.5 + hd * y * y)
            dis[pl.ds(off + i, 16)] = jnp.where(d > 0.0, y, 0.0)

    @pl.when(wid < NTILES - 1)
    def _():
        pltpu.sync_copy(x_hbm.at[pl.ds(base, RPT)], acc)

    @pl.when(wid == NTILES - 1)
    def _():
        pltpu.sync_copy(x_hbm.at[pl.ds(base, NROW_LAST)],
                        acc.at[pl.ds(0, NROW_LAST)])
        zf = jnp.zeros((16,), jnp.float32)

        @pl.loop(NROW_LAST, RPT)
        def _(r):
            @pl.loop(0, D, step=16)
            def _(j):
                acc[r, pl.ds(j, 16)] = zf

    def _scale_blk(rb, carry):
        r0 = rb * 16
        dv = dis[pl.ds(base + r0, 16)]
        for r in range(16):
            wb = jnp.full((16,), dv[r], jnp.float32)
            vals = [acc[r0 + r, pl.ds(j, 16)] for j in range(0, D, 16)]
            for k, j in enumerate(range(0, D, 16)):
                acc[r0 + r, pl.ds(j, 16)] = wb * vals[k]
        return carry

    lax.fori_loop(0, RPT // 16, _scale_blk, 0)

    lane = lax.broadcasted_iota(jnp.int32, (16,), 0)
    zi = jnp.zeros((16,), jnp.int32)

    @pl.loop(0, NCHUNKS)
    def _(c):
        pltpu.sync_copy(row_hbm.at[pl.ds(c * CHUNK, CHUNK)], rowb)
        pltpu.sync_copy(col_hbm.at[pl.ds(c * CHUNK, CHUNK)], colb)

        def scan_body(i, cnt):
            r16 = rowb[pl.ds(i * 16, 16)]
            c16 = colb[pl.ds(i * 16, 16)]
            m = (r16 >= base) & (r16 < base + RPT) & (r16 != c16)
            plsc.store_compressed(frloc.at[pl.ds(cnt, 16)], r16 - base,
                                  mask=m)
            plsc.store_compressed(fcol.at[pl.ds(cnt, 16)], c16, mask=m)
            pc = plsc.all_reduce_population_count(m)
            return cnt + jnp.max(pc)

        cnt = lax.fori_loop(0, CHUNK // 16, scan_body, jnp.int32(0))

        for t in range(G // 16):
            frloc[pl.ds(cnt + t * 16, 16)] = zi
            fcol[pl.ds(cnt + t * 16, 16)] = zi

        def grp_body(g, carry):
            goff = g * G
            handles = []
            for t in range(G // 16):
                cc16 = fcol[pl.ds(goff + t * 16, 16)]
                for r in range(16):
                    handles.append(pltpu.async_copy(
                        xpk_hbm.at[pl.ds(cc16[r], 1)],
                        stage_i.at[pl.ds(t * 16 + r, 1)], gsem))
            for h in handles:
                h.wait()

            def blk_body(b, bcarry):
                boff = goff + b * 16
                cc = fcol[pl.ds(boff, 16)]
                w = plsc.load_gather(dis, [cc])
                w = jnp.where((boff + lane) < cnt, w, 0.0)
                rl16 = frloc[pl.ds(boff, 16)]
                for r in range(16):
                    rloc = rl16[r]
                    wb = jnp.full((16,), w[r], jnp.float32)
                    srow = b * 16 + r
                    words = [stage_i[srow, pl.ds(j, 16)]
                             for j in range(0, D // 2, 16)]
                    for k, j in enumerate(range(0, D // 2, 16)):
                        lo = plsc.bitcast(words[k] << 16, jnp.float32)
                        hi = plsc.bitcast(words[k] & jnp.int32(-65536),
                                          jnp.float32)
                        plsc.addupdate(acc.at[rloc, pl.ds(j, 16)],
                                       wb * lo)
                        plsc.addupdate(acc.at[rloc, pl.ds(j + D // 2, 16)],
                                       wb * hi)
                return bcarry

            lax.fori_loop(0, G // 16, blk_body, 0)
            return carry

        ngrp = (cnt + G - 1) // G
        lax.fori_loop(0, ngrp, grp_body, 0)

    lax.fori_loop(0, RPT // 16, _scale_blk, 0)

    @pl.when(wid < NTILES - 1)
    def _():
        pltpu.sync_copy(acc, out_hbm.at[pl.ds(base, RPT)])

    @pl.when(wid == NTILES - 1)
    def _():
        pltpu.sync_copy(acc.at[pl.ds(0, NROW_LAST)],
                        out_hbm.at[pl.ds(base, NROW_LAST)])


@jax.jit
def kernel(x, edge_index):
    ei = edge_index.astype(jnp.int32)
    row = ei[0]
    col = ei[1]
    degp, xpk = _deg_kernel(row, x)
    return _prop_kernel(x, row, col, degp, xpk)


# speedup vs baseline: 2.1339x; 2.1339x over previous
import dataclasses
import functools

import jax
import jax.numpy as jnp
from jax import lax
from jax.experimental import pallas as pl
from jax.experimental.pallas import tpu as pltpu
from jax.experimental.pallas import tpu_sc as plsc

N = 10000
D = 256
E = 160000
NTILES = 32
RPT = 320
NPAD = NTILES * RPT
CHUNK = 1600
NCHUNKS = E // CHUNK
G = 64
ESL = E // NTILES
ESLPAD = 5008
NROW_LAST = N - (NTILES - 1) * RPT
SL = NPAD // 16

_mesh = plsc.VectorSubcoreMesh(core_axis_name="c", subcore_axis_name="s")

_cp = pltpu.CompilerParams()
if "needs_layout_passes" in pltpu.CompilerParams.__dataclass_fields__:
    _cp = dataclasses.replace(_cp, needs_layout_passes=False)


@functools.partial(
    pl.kernel,
    out_type=(
        jax.ShapeDtypeStruct((2, NPAD), jnp.float32),
        jax.ShapeDtypeStruct((NPAD, D // 2), jnp.int32),
    ),
    mesh=_mesh,
    scratch_types=[
        pltpu.VMEM((ESLPAD,), jnp.int32),
        pltpu.VMEM((NPAD,), jnp.float32),
        pltpu.VMEM_SHARED((16, NPAD), jnp.float32),
        pltpu.VMEM((SL,), jnp.float32),
        pltpu.VMEM((G, D), jnp.float32),
        pltpu.VMEM((G, D // 2), jnp.int32),
    ],
    compiler_params=_cp,
)
def _deg_kernel(row_hbm, x_hbm, degp_hbm, xpk_hbm, rows_v, hist_v, shared,
                red_v, stage, stage_i):
    cid = lax.axis_index("c")
    sid = lax.axis_index("s")
    wid = sid * 2 + cid

    def _pack_rows(nrows):
        def body(r, carry):
            for j in range(0, D // 2, 16):
                a = plsc.bitcast(stage[r, pl.ds(j, 16)], jnp.int32)
                b = plsc.bitcast(stage[r, pl.ds(j + D // 2, 16)],
                                 jnp.int32)
                a = (a + 0x8000) >> 16
                b = (b + 0x8000) & jnp.int32(-65536)
                stage_i[r, pl.ds(j, 16)] = (a & 0xFFFF) | b
            return carry

        lax.fori_loop(0, nrows, body, 0)

    @pl.loop(0, RPT, step=G)
    def _(gi):
        s0 = wid * RPT + gi

        @pl.when(s0 + G <= N)
        def _():
            pltpu.sync_copy(x_hbm.at[pl.ds(s0, G)], stage)
            _pack_rows(G)
            pltpu.sync_copy(stage_i, xpk_hbm.at[pl.ds(s0, G)])

        @pl.when((s0 < N) & (s0 + G > N))
        def _():
            nr = N % G
            pltpu.sync_copy(x_hbm.at[pl.ds(s0, nr)],
                            stage.at[pl.ds(0, nr)])
            _pack_rows(nr)
            pltpu.sync_copy(stage_i.at[pl.ds(0, nr)],
                            xpk_hbm.at[pl.ds(s0, nr)])

    zf = jnp.zeros((16,), jnp.float32)

    @pl.loop(0, NPAD, step=16)
    def _(i):
        hist_v[pl.ds(i, 16)] = zf

    pltpu.sync_copy(row_hbm.at[pl.ds(wid * ESL, ESL)],
                    rows_v.at[pl.ds(0, ESL)])

    ones = jnp.ones((16,), jnp.float32)

    ESL_FULL = (ESL // 16) * 16

    @pl.loop(0, ESL_FULL, step=16)
    def _(i):
        idx = rows_v[pl.ds(i, 16)]
        plsc.addupdate_scatter(hist_v, [idx], ones)

    lane = lax.broadcasted_iota(jnp.int32, (16,), 0)
    tidx = rows_v[pl.ds(ESL_FULL, 16)]
    plsc.addupdate_scatter(hist_v, [tidx], ones, mask=lane < (ESL - ESL_FULL))

    pltpu.sync_copy(hist_v, shared.at[sid])
    plsc.subcore_barrier()

    @pl.loop(0, SL, step=16)
    def _(i):
        red_v[pl.ds(i, 16)] = zf

    @pl.loop(0, 16)
    def _(t):
        pltpu.sync_copy(shared.at[t, pl.ds(sid * SL, SL)],
                        hist_v.at[pl.ds(0, SL)])

        @pl.loop(0, SL, step=16)
        def _(i):
            red_v[pl.ds(i, 16)] = red_v[pl.ds(i, 16)] + hist_v[pl.ds(i, 16)]

    pltpu.sync_copy(red_v, degp_hbm.at[cid, pl.ds(sid * SL, SL)])


@functools.partial(
    pl.kernel,
    out_type=jax.ShapeDtypeStruct((N, D), jnp.float32),
    mesh=_mesh,
    scratch_types=[
        pltpu.VMEM((RPT, D), jnp.float32),
        pltpu.VMEM((NPAD,), jnp.float32),
        pltpu.VMEM((CHUNK,), jnp.int32),
        pltpu.VMEM((CHUNK,), jnp.int32),
        pltpu.VMEM((CHUNK + G,), jnp.int32),
        pltpu.VMEM((CHUNK + G,), jnp.int32),
        pltpu.VMEM((G,), jnp.int32),
        pltpu.VMEM((G, D // 2), jnp.int32),
        pltpu.VMEM((1024,), jnp.float32),
        pltpu.VMEM((1024,), jnp.float32),
        pltpu.SemaphoreType.DMA,
    ],
    compiler_params=_cp,
)
def _prop_kernel(x_hbm, row_hbm, col_hbm, degp_hbm, xpk_hbm, out_hbm,
                 acc, dis, rowb, colb, frloc, fcol, idxb,
                 stage_i, tmp0, tmp1, gsem):
    cid = lax.axis_index("c")
    sid = lax.axis_index("s")
    wid = sid * 2 + cid
    base = wid * RPT

    @pl.loop(0, NPAD, step=1024)
    def _(off):
        pltpu.sync_copy(degp_hbm.at[0, pl.ds(off, 1024)], tmp0)
        pltpu.sync_copy(degp_hbm.at[1, pl.ds(off, 1024)], tmp1)

        @pl.loop(0, 1024, step=16)
        def _(i):
            d = tmp0[pl.ds(i, 16)] + tmp1[pl.ds(i, 16)]
            bits = plsc.bitcast(d, jnp.int32)
            bits = 0x5F3759DF - (bits >> 1)
            y = plsc.bitcast(bits, jnp.float32)
            hd = -0.5 * d
            y = y * (1.5 + hd * y * y)
            y = y * (1.5 + hd * y * y)
            y = y * (1.5 + hd * y * y)
            dis[pl.ds(off + i, 16)] = jnp.where(d > 0.0, y, 0.0)

    @pl.when(wid < NTILES - 1)
    def _():
        pltpu.sync_copy(x_hbm.at[pl.ds(base, RPT)], acc)

    @pl.when(wid == NTILES - 1)
    def _():
        pltpu.sync_copy(x_hbm.at[pl.ds(base, NROW_LAST)],
                        acc.at[pl.ds(0, NROW_LAST)])
        zf = jnp.zeros((16,), jnp.float32)

        @pl.loop(NROW_LAST, RPT)
        def _(r):
            @pl.loop(0, D, step=16)
            def _(j):
                acc[r, pl.ds(j, 16)] = zf

    def _scale_blk(rb, carry):
        r0 = rb * 16
        dv = dis[pl.ds(base + r0, 16)]
        for r in range(16):
            wb = jnp.full((16,), dv[r], jnp.float32)
            vals = [acc[r0 + r, pl.ds(j, 16)] for j in range(0, D, 16)]
            for k, j in enumerate(range(0, D, 16)):
                acc[r0 + r, pl.ds(j, 16)] = wb * vals[k]
        return carry

    lax.fori_loop(0, RPT // 16, _scale_blk, 0)

    lane = lax.broadcasted_iota(jnp.int32, (16,), 0)
    zi = jnp.zeros((16,), jnp.int32)

    @pl.loop(0, NCHUNKS)
    def _(c):
        pltpu.sync_copy(row_hbm.at[pl.ds(c * CHUNK, CHUNK)], rowb)
        pltpu.sync_copy(col_hbm.at[pl.ds(c * CHUNK, CHUNK)], colb)

        def scan_body(i, cnt):
            r16 = rowb[pl.ds(i * 16, 16)]
            c16 = colb[pl.ds(i * 16, 16)]
            m = (r16 >= base) & (r16 < base + RPT) & (r16 != c16)
            plsc.store_compressed(frloc.at[pl.ds(cnt, 16)], r16 - base,
                                  mask=m)
            plsc.store_compressed(fcol.at[pl.ds(cnt, 16)], c16, mask=m)
            pc = plsc.all_reduce_population_count(m)
            return cnt + jnp.max(pc)

        cnt = lax.fori_loop(0, CHUNK // 16, scan_body, jnp.int32(0))

        for t in range(G // 16):
            frloc[pl.ds(cnt + t * 16, 16)] = zi
            fcol[pl.ds(cnt + t * 16, 16)] = zi

        def grp_body(g, carry):
            goff = g * G
            rem = cnt - goff
            ns = jnp.minimum((rem + 15) // 16, G // 16)

            def issue(s, c2):
                pltpu.async_copy(
                    xpk_hbm.at[fcol.at[pl.ds(goff + s * 16, 16)]],
                    stage_i.at[pl.ds(s * 16, 16)], gsem)
                return c2

            lax.fori_loop(0, ns, issue, 0)

            def drain(s, c2):
                pltpu.make_async_copy(
                    xpk_hbm.at[fcol.at[pl.ds(goff + s * 16, 16)]],
                    stage_i.at[pl.ds(s * 16, 16)], gsem).wait()
                return c2

            lax.fori_loop(0, ns, drain, 0)

            def blk_body(b, bcarry):
                boff = goff + b * 16
                cc = fcol[pl.ds(boff, 16)]
                w = plsc.load_gather(dis, [cc])
                w = jnp.where((boff + lane) < cnt, w, 0.0)
                rl16 = frloc[pl.ds(boff, 16)]
                for r in range(16):
                    rloc = rl16[r]
                    wb = jnp.full((16,), w[r], jnp.float32)
                    srow = b * 16 + r
                    words = [stage_i[srow, pl.ds(j, 16)]
                             for j in range(0, D // 2, 16)]
                    for k, j in enumerate(range(0, D // 2, 16)):
                        lo = plsc.bitcast(words[k] << 16, jnp.float32)
                        hi = plsc.bitcast(words[k] & jnp.int32(-65536),
                                          jnp.float32)
                        plsc.addupdate(acc.at[rloc, pl.ds(j, 16)],
                                       wb * lo)
                        plsc.addupdate(acc.at[rloc, pl.ds(j + D // 2, 16)],
                                       wb * hi)
                return bcarry

            lax.fori_loop(0, ns, blk_body, 0)
            return carry

        ngrp = (cnt + G - 1) // G
        lax.fori_loop(0, ngrp, grp_body, 0)

    lax.fori_loop(0, RPT // 16, _scale_blk, 0)

    @pl.when(wid < NTILES - 1)
    def _():
        pltpu.sync_copy(acc, out_hbm.at[pl.ds(base, RPT)])

    @pl.when(wid == NTILES - 1)
    def _():
        pltpu.sync_copy(acc.at[pl.ds(0, NROW_LAST)],
                        out_hbm.at[pl.ds(base, NROW_LAST)])


@jax.jit
def kernel(x, edge_index):
    ei = edge_index.astype(jnp.int32)
    row = ei[0]
    col = ei[1]
    degp, xpk = _deg_kernel(row, x)
    return _prop_kernel(x, row, col, degp, xpk)


# speedup vs baseline: 3.5549x; 1.6659x over previous
import dataclasses
import functools

import jax
import jax.numpy as jnp
from jax import lax
from jax.experimental import pallas as pl
from jax.experimental.pallas import tpu as pltpu
from jax.experimental.pallas import tpu_sc as plsc

N = 10000
D = 256
E = 160000
NTILES = 32
RPT = 320
NPAD = NTILES * RPT
CHUNK = 3200
NCHUNKS = E // CHUNK
G = 128
GP = 64
ESL = E // NTILES
ESLPAD = 5008
NROW_LAST = N - (NTILES - 1) * RPT
SL = NPAD // 16

_mesh = plsc.VectorSubcoreMesh(core_axis_name="c", subcore_axis_name="s")

_cp = pltpu.CompilerParams()
if "needs_layout_passes" in pltpu.CompilerParams.__dataclass_fields__:
    _cp = dataclasses.replace(_cp, needs_layout_passes=False)


@functools.partial(
    pl.kernel,
    out_type=(
        jax.ShapeDtypeStruct((2, NPAD), jnp.float32),
        jax.ShapeDtypeStruct((NPAD, D // 2), jnp.int32),
    ),
    mesh=_mesh,
    scratch_types=[
        pltpu.VMEM((ESLPAD,), jnp.int32),
        pltpu.VMEM((NPAD,), jnp.float32),
        pltpu.VMEM_SHARED((16, NPAD), jnp.float32),
        pltpu.VMEM((SL,), jnp.float32),
        pltpu.VMEM((GP, D), jnp.float32),
        pltpu.VMEM((GP, D // 2), jnp.int32),
    ],
    compiler_params=_cp,
)
def _deg_kernel(row_hbm, x_hbm, degp_hbm, xpk_hbm, rows_v, hist_v, shared,
                red_v, stage, stage_i):
    cid = lax.axis_index("c")
    sid = lax.axis_index("s")
    wid = sid * 2 + cid

    def _pack_rows(nrows):
        def body(r, carry):
            for j in range(0, D // 2, 16):
                a = plsc.bitcast(stage[r, pl.ds(j, 16)], jnp.int32)
                b = plsc.bitcast(stage[r, pl.ds(j + D // 2, 16)],
                                 jnp.int32)
                a = (a + 0x8000) >> 16
                b = (b + 0x8000) & jnp.int32(-65536)
                stage_i[r, pl.ds(j, 16)] = (a & 0xFFFF) | b
            return carry

        lax.fori_loop(0, nrows, body, 0)

    @pl.loop(0, RPT, step=GP)
    def _(gi):
        s0 = wid * RPT + gi

        @pl.when(s0 + GP <= N)
        def _():
            pltpu.sync_copy(x_hbm.at[pl.ds(s0, GP)], stage)
            _pack_rows(GP)
            pltpu.sync_copy(stage_i, xpk_hbm.at[pl.ds(s0, GP)])

        @pl.when((s0 < N) & (s0 + GP > N))
        def _():
            nr = N % GP
            pltpu.sync_copy(x_hbm.at[pl.ds(s0, nr)],
                            stage.at[pl.ds(0, nr)])
            _pack_rows(nr)
            pltpu.sync_copy(stage_i.at[pl.ds(0, nr)],
                            xpk_hbm.at[pl.ds(s0, nr)])

    zf = jnp.zeros((16,), jnp.float32)

    @pl.loop(0, NPAD, step=16)
    def _(i):
        hist_v[pl.ds(i, 16)] = zf

    pltpu.sync_copy(row_hbm.at[pl.ds(wid * ESL, ESL)],
                    rows_v.at[pl.ds(0, ESL)])

    ones = jnp.ones((16,), jnp.float32)

    ESL_FULL = (ESL // 16) * 16

    @pl.loop(0, ESL_FULL, step=16)
    def _(i):
        idx = rows_v[pl.ds(i, 16)]
        plsc.addupdate_scatter(hist_v, [idx], ones)

    lane = lax.broadcasted_iota(jnp.int32, (16,), 0)
    tidx = rows_v[pl.ds(ESL_FULL, 16)]
    plsc.addupdate_scatter(hist_v, [tidx], ones, mask=lane < (ESL - ESL_FULL))

    pltpu.sync_copy(hist_v, shared.at[sid])
    plsc.subcore_barrier()

    @pl.loop(0, SL, step=16)
    def _(i):
        red_v[pl.ds(i, 16)] = zf

    @pl.loop(0, 16)
    def _(t):
        pltpu.sync_copy(shared.at[t, pl.ds(sid * SL, SL)],
                        hist_v.at[pl.ds(0, SL)])

        @pl.loop(0, SL, step=16)
        def _(i):
            red_v[pl.ds(i, 16)] = red_v[pl.ds(i, 16)] + hist_v[pl.ds(i, 16)]

    pltpu.sync_copy(red_v, degp_hbm.at[cid, pl.ds(sid * SL, SL)])


@functools.partial(
    pl.kernel,
    out_type=jax.ShapeDtypeStruct((N, D), jnp.float32),
    mesh=_mesh,
    scratch_types=[
        pltpu.VMEM((RPT, D), jnp.float32),
        pltpu.VMEM((NPAD,), jnp.float32),
        pltpu.VMEM((CHUNK,), jnp.int32),
        pltpu.VMEM((CHUNK,), jnp.int32),
        pltpu.VMEM((CHUNK + G,), jnp.int32),
        pltpu.VMEM((CHUNK + G,), jnp.int32),
        pltpu.VMEM((G,), jnp.int32),
        pltpu.VMEM((G, D // 2), jnp.int32),
        pltpu.VMEM((1024,), jnp.float32),
        pltpu.VMEM((1024,), jnp.float32),
        pltpu.SemaphoreType.DMA,
    ],
    compiler_params=_cp,
)
def _prop_kernel(x_hbm, row_hbm, col_hbm, degp_hbm, xpk_hbm, out_hbm,
                 acc, dis, rowb, colb, frloc, fcol, idxb,
                 stage_i, tmp0, tmp1, gsem):
    cid = lax.axis_index("c")
    sid = lax.axis_index("s")
    wid = sid * 2 + cid
    base = wid * RPT

    @pl.loop(0, NPAD, step=1024)
    def _(off):
        pltpu.sync_copy(degp_hbm.at[0, pl.ds(off, 1024)], tmp0)
        pltpu.sync_copy(degp_hbm.at[1, pl.ds(off, 1024)], tmp1)

        @pl.loop(0, 1024, step=16)
        def _(i):
            d = tmp0[pl.ds(i, 16)] + tmp1[pl.ds(i, 16)]
            bits = plsc.bitcast(d, jnp.int32)
            bits = 0x5F3759DF - (bits >> 1)
            y = plsc.bitcast(bits, jnp.float32)
            hd = -0.5 * d
            y = y * (1.5 + hd * y * y)
            y = y * (1.5 + hd * y * y)
            y = y * (1.5 + hd * y * y)
            dis[pl.ds(off + i, 16)] = jnp.where(d > 0.0, y, 0.0)

    @pl.when(wid < NTILES - 1)
    def _():
        pltpu.sync_copy(x_hbm.at[pl.ds(base, RPT)], acc)

    @pl.when(wid == NTILES - 1)
    def _():
        pltpu.sync_copy(x_hbm.at[pl.ds(base, NROW_LAST)],
                        acc.at[pl.ds(0, NROW_LAST)])
        zf = jnp.zeros((16,), jnp.float32)

        @pl.loop(NROW_LAST, RPT)
        def _(r):
            @pl.loop(0, D, step=16)
            def _(j):
                acc[r, pl.ds(j, 16)] = zf

    def _scale_blk(rb, carry):
        r0 = rb * 16
        dv = dis[pl.ds(base + r0, 16)]
        for r in range(16):
            wb = jnp.full((16,), dv[r], jnp.float32)
            vals = [acc[r0 + r, pl.ds(j, 16)] for j in range(0, D, 16)]
            for k, j in enumerate(range(0, D, 16)):
                acc[r0 + r, pl.ds(j, 16)] = wb * vals[k]
        return carry

    lax.fori_loop(0, RPT // 16, _scale_blk, 0)

    lane = lax.broadcasted_iota(jnp.int32, (16,), 0)
    zi = jnp.zeros((16,), jnp.int32)

    @pl.loop(0, NCHUNKS)
    def _(c):
        pltpu.sync_copy(row_hbm.at[pl.ds(c * CHUNK, CHUNK)], rowb)
        pltpu.sync_copy(col_hbm.at[pl.ds(c * CHUNK, CHUNK)], colb)

        def scan_body(i, cnt):
            r16 = rowb[pl.ds(i * 16, 16)]
            c16 = colb[pl.ds(i * 16, 16)]
            m = (r16 >= base) & (r16 < base + RPT) & (r16 != c16)
            plsc.store_compressed(frloc.at[pl.ds(cnt, 16)], r16 - base,
                                  mask=m)
            plsc.store_compressed(fcol.at[pl.ds(cnt, 16)], c16, mask=m)
            pc = plsc.all_reduce_population_count(m)
            return cnt + jnp.max(pc)

        cnt = lax.fori_loop(0, CHUNK // 16, scan_body, jnp.int32(0))

        for t in range(G // 16):
            frloc[pl.ds(cnt + t * 16, 16)] = zi
            fcol[pl.ds(cnt + t * 16, 16)] = zi

        def grp_body(g, carry):
            goff = g * G
            rem = cnt - goff
            ns = jnp.minimum((rem + 15) // 16, G // 16)

            def issue(s, c2):
                pltpu.async_copy(
                    xpk_hbm.at[fcol.at[pl.ds(goff + s * 16, 16)]],
                    stage_i.at[pl.ds(s * 16, 16)], gsem)
                return c2

            lax.fori_loop(0, ns, issue, 0)

            def drain(s, c2):
                pltpu.make_async_copy(
                    xpk_hbm.at[fcol.at[pl.ds(goff + s * 16, 16)]],
                    stage_i.at[pl.ds(s * 16, 16)], gsem).wait()
                return c2

            lax.fori_loop(0, ns, drain, 0)

            def blk_body(b, bcarry):
                boff = goff + b * 16
                cc = fcol[pl.ds(boff, 16)]
                w = plsc.load_gather(dis, [cc])
                w = jnp.where((boff + lane) < cnt, w, 0.0)
                rl16 = frloc[pl.ds(boff, 16)]
                for r in range(16):
                    rloc = rl16[r]
                    wb = jnp.full((16,), w[r], jnp.float32)
                    srow = b * 16 + r
                    words = [stage_i[srow, pl.ds(j, 16)]
                             for j in range(0, D // 2, 16)]
                    for k, j in enumerate(range(0, D // 2, 16)):
                        lo = plsc.bitcast(words[k] << 16, jnp.float32)
                        hi = plsc.bitcast(words[k] & jnp.int32(-65536),
                                          jnp.float32)
                        plsc.addupdate(acc.at[rloc, pl.ds(j, 16)],
                                       wb * lo)
                        plsc.addupdate(acc.at[rloc, pl.ds(j + D // 2, 16)],
                                       wb * hi)
                return bcarry

            lax.fori_loop(0, ns, blk_body, 0)
            return carry

        ngrp = (cnt + G - 1) // G
        lax.fori_loop(0, ngrp, grp_body, 0)

    lax.fori_loop(0, RPT // 16, _scale_blk, 0)

    @pl.when(wid < NTILES - 1)
    def _():
        pltpu.sync_copy(acc, out_hbm.at[pl.ds(base, RPT)])

    @pl.when(wid == NTILES - 1)
    def _():
        pltpu.sync_copy(acc.at[pl.ds(0, NROW_LAST)],
                        out_hbm.at[pl.ds(base, NROW_LAST)])


@jax.jit
def kernel(x, edge_index):
    ei = edge_index.astype(jnp.int32)
    row = ei[0]
    col = ei[1]
    degp, xpk = _deg_kernel(row, x)
    return _prop_kernel(x, row, col, degp, xpk)


# speedup vs baseline: 4.1662x; 1.1720x over previous
import dataclasses
import functools

import jax
import jax.numpy as jnp
from jax import lax
from jax.experimental import pallas as pl
from jax.experimental.pallas import tpu as pltpu
from jax.experimental.pallas import tpu_sc as plsc

N = 10000
D = 256
E = 160000
NTILES = 32
RPT = 320
NPAD = NTILES * RPT
CHUNK = 4000
NCHUNKS = E // CHUNK
G = 160
GP = 64
ESL = E // NTILES
ESLPAD = 5008
NROW_LAST = N - (NTILES - 1) * RPT
SL = NPAD // 16

_mesh = plsc.VectorSubcoreMesh(core_axis_name="c", subcore_axis_name="s")

_cp = pltpu.CompilerParams()
if "needs_layout_passes" in pltpu.CompilerParams.__dataclass_fields__:
    _cp = dataclasses.replace(_cp, needs_layout_passes=False)


@functools.partial(
    pl.kernel,
    out_type=(
        jax.ShapeDtypeStruct((2, NPAD), jnp.float32),
        jax.ShapeDtypeStruct((NPAD, D // 2), jnp.int32),
    ),
    mesh=_mesh,
    scratch_types=[
        pltpu.VMEM((ESLPAD,), jnp.int32),
        pltpu.VMEM((NPAD,), jnp.float32),
        pltpu.VMEM_SHARED((16, NPAD), jnp.float32),
        pltpu.VMEM((SL,), jnp.float32),
        pltpu.VMEM((GP, D), jnp.float32),
        pltpu.VMEM((GP, D // 2), jnp.int32),
    ],
    compiler_params=_cp,
)
def _deg_kernel(row_hbm, x_hbm, degp_hbm, xpk_hbm, rows_v, hist_v, shared,
                red_v, stage, stage_i):
    cid = lax.axis_index("c")
    sid = lax.axis_index("s")
    wid = sid * 2 + cid

    def _pack_rows(nrows):
        def body(r, carry):
            for j in range(0, D // 2, 16):
                a = plsc.bitcast(stage[r, pl.ds(j, 16)], jnp.int32)
                b = plsc.bitcast(stage[r, pl.ds(j + D // 2, 16)],
                                 jnp.int32)
                a = (a + 0x8000) >> 16
                b = (b + 0x8000) & jnp.int32(-65536)
                stage_i[r, pl.ds(j, 16)] = (a & 0xFFFF) | b
            return carry

        lax.fori_loop(0, nrows, body, 0)

    @pl.loop(0, RPT, step=GP)
    def _(gi):
        s0 = wid * RPT + gi

        @pl.when(s0 + GP <= N)
        def _():
            pltpu.sync_copy(x_hbm.at[pl.ds(s0, GP)], stage)
            _pack_rows(GP)
            pltpu.sync_copy(stage_i, xpk_hbm.at[pl.ds(s0, GP)])

        @pl.when((s0 < N) & (s0 + GP > N))
        def _():
            nr = N % GP
            pltpu.sync_copy(x_hbm.at[pl.ds(s0, nr)],
                            stage.at[pl.ds(0, nr)])
            _pack_rows(nr)
            pltpu.sync_copy(stage_i.at[pl.ds(0, nr)],
                            xpk_hbm.at[pl.ds(s0, nr)])

    zf = jnp.zeros((16,), jnp.float32)

    @pl.loop(0, NPAD, step=16)
    def _(i):
        hist_v[pl.ds(i, 16)] = zf

    pltpu.sync_copy(row_hbm.at[pl.ds(wid * ESL, ESL)],
                    rows_v.at[pl.ds(0, ESL)])

    ones = jnp.ones((16,), jnp.float32)

    ESL_FULL = (ESL // 16) * 16

    @pl.loop(0, ESL_FULL, step=16)
    def _(i):
        idx = rows_v[pl.ds(i, 16)]
        plsc.addupdate_scatter(hist_v, [idx], ones)

    lane = lax.broadcasted_iota(jnp.int32, (16,), 0)
    tidx = rows_v[pl.ds(ESL_FULL, 16)]
    plsc.addupdate_scatter(hist_v, [tidx], ones, mask=lane < (ESL - ESL_FULL))

    pltpu.sync_copy(hist_v, shared.at[sid])
    plsc.subcore_barrier()

    @pl.loop(0, SL, step=16)
    def _(i):
        red_v[pl.ds(i, 16)] = zf

    @pl.loop(0, 16)
    def _(t):
        pltpu.sync_copy(shared.at[t, pl.ds(sid * SL, SL)],
                        hist_v.at[pl.ds(0, SL)])

        @pl.loop(0, SL, step=16)
        def _(i):
            red_v[pl.ds(i, 16)] = red_v[pl.ds(i, 16)] + hist_v[pl.ds(i, 16)]

    pltpu.sync_copy(red_v, degp_hbm.at[cid, pl.ds(sid * SL, SL)])


@functools.partial(
    pl.kernel,
    out_type=jax.ShapeDtypeStruct((N, D), jnp.float32),
    mesh=_mesh,
    scratch_types=[
        pltpu.VMEM((RPT, D), jnp.float32),
        pltpu.VMEM((NPAD,), jnp.float32),
        pltpu.VMEM((CHUNK,), jnp.int32),
        pltpu.VMEM((CHUNK,), jnp.int32),
        pltpu.VMEM((CHUNK + G,), jnp.int32),
        pltpu.VMEM((CHUNK + G,), jnp.int32),
        pltpu.VMEM((G, D // 2), jnp.int32),
        pltpu.VMEM((512,), jnp.float32),
        pltpu.VMEM((512,), jnp.float32),
        pltpu.SemaphoreType.DMA,
    ],
    compiler_params=_cp,
)
def _prop_kernel(x_hbm, row_hbm, col_hbm, degp_hbm, xpk_hbm, out_hbm,
                 acc, dis, rowb, colb, frloc, fcol,
                 stage_i, tmp0, tmp1, gsem):
    cid = lax.axis_index("c")
    sid = lax.axis_index("s")
    wid = sid * 2 + cid
    base = wid * RPT

    @pl.loop(0, NPAD, step=512)
    def _(off):
        pltpu.sync_copy(degp_hbm.at[0, pl.ds(off, 512)], tmp0)
        pltpu.sync_copy(degp_hbm.at[1, pl.ds(off, 512)], tmp1)

        @pl.loop(0, 512, step=16)
        def _(i):
            d = tmp0[pl.ds(i, 16)] + tmp1[pl.ds(i, 16)]
            bits = plsc.bitcast(d, jnp.int32)
            bits = 0x5F3759DF - (bits >> 1)
            y = plsc.bitcast(bits, jnp.float32)
            hd = -0.5 * d
            y = y * (1.5 + hd * y * y)
            y = y * (1.5 + hd * y * y)
            y = y * (1.5 + hd * y * y)
            dis[pl.ds(off + i, 16)] = jnp.where(d > 0.0, y, 0.0)

    @pl.when(wid < NTILES - 1)
    def _():
        pltpu.sync_copy(x_hbm.at[pl.ds(base, RPT)], acc)

    @pl.when(wid == NTILES - 1)
    def _():
        pltpu.sync_copy(x_hbm.at[pl.ds(base, NROW_LAST)],
                        acc.at[pl.ds(0, NROW_LAST)])
        zf = jnp.zeros((16,), jnp.float32)

        @pl.loop(NROW_LAST, RPT)
        def _(r):
            @pl.loop(0, D, step=16)
            def _(j):
                acc[r, pl.ds(j, 16)] = zf

    def _scale_blk(rb, carry):
        r0 = rb * 16
        dv = dis[pl.ds(base + r0, 16)]
        for r in range(16):
            wb = jnp.full((16,), dv[r], jnp.float32)
            vals = [acc[r0 + r, pl.ds(j, 16)] for j in range(0, D, 16)]
            for k, j in enumerate(range(0, D, 16)):
                acc[r0 + r, pl.ds(j, 16)] = wb * vals[k]
        return carry

    lax.fori_loop(0, RPT // 16, _scale_blk, 0)

    lane = lax.broadcasted_iota(jnp.int32, (16,), 0)
    zi = jnp.zeros((16,), jnp.int32)

    @pl.loop(0, NCHUNKS)
    def _(c):
        pltpu.sync_copy(row_hbm.at[pl.ds(c * CHUNK, CHUNK)], rowb)
        pltpu.sync_copy(col_hbm.at[pl.ds(c * CHUNK, CHUNK)], colb)

        def scan_body(i, cnt):
            r16 = rowb[pl.ds(i * 16, 16)]
            c16 = colb[pl.ds(i * 16, 16)]
            m = (r16 >= base) & (r16 < base + RPT) & (r16 != c16)
            plsc.store_compressed(frloc.at[pl.ds(cnt, 16)], r16 - base,
                                  mask=m)
            plsc.store_compressed(fcol.at[pl.ds(cnt, 16)], c16, mask=m)
            pc = plsc.all_reduce_population_count(m)
            return cnt + jnp.max(pc)

        cnt = lax.fori_loop(0, CHUNK // 16, scan_body, jnp.int32(0))

        for t in range(G // 16):
            frloc[pl.ds(cnt + t * 16, 16)] = zi
            fcol[pl.ds(cnt + t * 16, 16)] = zi

        def grp_body(g, carry):
            goff = g * G
            rem = cnt - goff
            ns = jnp.minimum((rem + 15) // 16, G // 16)

            def issue(s, c2):
                pltpu.async_copy(
                    xpk_hbm.at[fcol.at[pl.ds(goff + s * 16, 16)]],
                    stage_i.at[pl.ds(s * 16, 16)], gsem)
                return c2

            lax.fori_loop(0, ns, issue, 0)

            def drain(s, c2):
                pltpu.make_async_copy(
                    xpk_hbm.at[fcol.at[pl.ds(goff + s * 16, 16)]],
                    stage_i.at[pl.ds(s * 16, 16)], gsem).wait()
                return c2

            lax.fori_loop(0, ns, drain, 0)

            def blk_body(b, bcarry):
                boff = goff + b * 16
                cc = fcol[pl.ds(boff, 16)]
                w = plsc.load_gather(dis, [cc])
                w = jnp.where((boff + lane) < cnt, w, 0.0)
                rl16 = frloc[pl.ds(boff, 16)]
                for r in range(16):
                    rloc = rl16[r]
                    wb = jnp.full((16,), w[r], jnp.float32)
                    srow = b * 16 + r
                    words = [stage_i[srow, pl.ds(j, 16)]
                             for j in range(0, D // 2, 16)]
                    for k, j in enumerate(range(0, D // 2, 16)):
                        lo = plsc.bitcast(words[k] << 16, jnp.float32)
                        hi = plsc.bitcast(words[k] & jnp.int32(-65536),
                                          jnp.float32)
                        plsc.addupdate(acc.at[rloc, pl.ds(j, 16)],
                                       wb * lo)
                        plsc.addupdate(acc.at[rloc, pl.ds(j + D // 2, 16)],
                                       wb * hi)
                return bcarry

            lax.fori_loop(0, ns, blk_body, 0)
            return carry

        ngrp = (cnt + G - 1) // G
        lax.fori_loop(0, ngrp, grp_body, 0)

    lax.fori_loop(0, RPT // 16, _scale_blk, 0)

    @pl.when(wid < NTILES - 1)
    def _():
        pltpu.sync_copy(acc, out_hbm.at[pl.ds(base, RPT)])

    @pl.when(wid == NTILES - 1)
    def _():
        pltpu.sync_copy(acc.at[pl.ds(0, NROW_LAST)],
                        out_hbm.at[pl.ds(base, NROW_LAST)])


@jax.jit
def kernel(x, edge_index):
    ei = edge_index.astype(jnp.int32)
    row = ei[0]
    col = ei[1]
    degp, xpk = _deg_kernel(row, x)
    return _prop_kernel(x, row, col, degp, xpk)
